# baseline (speedup 1.0000x reference)
import functools

import jax
import jax.numpy as jnp
from jax import lax
from jax.experimental import pallas as pl
from jax.experimental.pallas import tpu as pltpu
from jax.experimental.pallas import tpu_sc as plsc

D = 64
L = 128
K = 4
TBLK = 8192


def _tc_transpose(wt):
    d, v = wt.shape
    grid = (v + TBLK - 1) // TBLK

    def body(in_ref, out_ref):
        a = in_ref[...].T
        out_ref[:, 0:d] = a[0 : TBLK // 2, :]
        out_ref[:, d : 2 * d] = a[TBLK // 2 : TBLK, :]

    return pl.pallas_call(
        body,
        grid=(grid,),
        in_specs=[pl.BlockSpec((d, TBLK), lambda g: (0, g))],
        out_specs=pl.BlockSpec((TBLK // 2, 2 * d), lambda g: (g, 0)),
        out_shape=jax.ShapeDtypeStruct((grid * TBLK // 2, 2 * d), jnp.float32),
    )(wt)


def _make_lookup(n_rows: int, vocab_pad: int):
    info = plsc.get_sparse_core_info()
    nc, ns = info.num_cores, info.num_subcores
    nw = nc * ns
    rows_per_w = n_rows // nw
    iters = rows_per_w // K
    assert rows_per_w % K == 0

    mesh = plsc.VectorSubcoreMesh(core_axis_name="c", subcore_axis_name="s")

    @functools.partial(
        pl.kernel,
        mesh=mesh,
        out_type=jax.ShapeDtypeStruct((n_rows * L, D), jnp.float32),
        scratch_types=[
            pltpu.VMEM((K, L), jnp.int32),
            pltpu.VMEM((K * L, D), jnp.float32),
            pltpu.SemaphoreType.DMA,
        ],
        compiler_params=pltpu.CompilerParams(use_tc_tiling_on_sc=False),
    )
    def lookup(idx_hbm, table_hbm, out_hbm, idx_v, rows_v, sem):
        wid = lax.axis_index("s") * nc + lax.axis_index("c")
        row_base = wid * rows_per_w

        def body(g, carry):
            base = row_base + g * K
            pltpu.sync_copy(idx_hbm.at[pl.ds(base, K)], idx_v)
            copies = [
                pltpu.async_copy(
                    table_hbm.at[idx_v.at[j]],
                    rows_v.at[pl.ds(j * L, L)],
                    sem,
                )
                for j in range(K)
            ]
            for cp in copies:
                cp.wait()
            pltpu.sync_copy(rows_v, out_hbm.at[pl.ds(base * L, K * L)])
            return carry

        lax.fori_loop(0, iters, body, 0)

    return lookup


def _tc_to_tiles(lin, nh, nb):
    tg = 4
    lin4 = lin.reshape(nh, nb, D, 2 * D)

    def body(in_ref, out_ref):
        for u in range(tg):
            a = in_ref[0, u]
            for par in range(2):
                t = a[:, par * D : (par + 1) * D].T
                out_ref[0, :, u, :, par * D : (par + 1) * D] = t.reshape(8, 8, D)

    return pl.pallas_call(
        body,
        grid=(nh, nb // tg),
        in_specs=[pl.BlockSpec((1, tg, D, 2 * D), lambda h, t: (h, t, 0, 0))],
        out_specs=pl.BlockSpec((1, 8, tg, 8, L), lambda h, t: (h, 0, t, 0, 0)),
        out_shape=jax.ShapeDtypeStruct((nh, 8, nb, 8, L), jnp.float32),
    )(lin4)


def kernel(x, word_vectors):
    b, h = x.shape
    vocab, d = word_vectors.shape
    n = b * h
    nb = b // L
    half = TBLK // 2
    xi = x.astype(jnp.int32)
    remapped = (xi // TBLK) * TBLK + 2 * (xi % half) + (xi // half) % 2
    xperm = (
        remapped.T.reshape(h, nb, 2, D)
        .transpose(0, 1, 3, 2)
        .reshape(n // L, L)
    )
    wlin = _tc_transpose(word_vectors.T)
    vocab_pad = wlin.shape[0] * 2
    out = _make_lookup(n // L, vocab_pad)(xperm, wlin.reshape(vocab_pad, d))
    out5 = _tc_to_tiles(out.reshape(n // 2, 2 * d), h, nb)
    return out5.transpose(2, 4, 0, 1, 3).reshape(b, h, d)


# speedup vs baseline: 1.0626x; 1.0626x over previous
import functools

import jax
import jax.numpy as jnp
from jax import lax
from jax.experimental import pallas as pl
from jax.experimental.pallas import tpu as pltpu
from jax.experimental.pallas import tpu_sc as plsc

D = 64
L = 128
K = 4
TBLK = 8192


def _tc_transpose(wt):
    d, v = wt.shape
    grid = (v + TBLK - 1) // TBLK

    def body(in_ref, out_ref):
        a = in_ref[...].T
        out_ref[:, 0:d] = a[0 : TBLK // 2, :]
        out_ref[:, d : 2 * d] = a[TBLK // 2 : TBLK, :]

    return pl.pallas_call(
        body,
        grid=(grid,),
        in_specs=[pl.BlockSpec((d, TBLK), lambda g: (0, g))],
        out_specs=pl.BlockSpec((TBLK // 2, 2 * d), lambda g: (g, 0)),
        out_shape=jax.ShapeDtypeStruct((grid * TBLK // 2, 2 * d), jnp.float32),
    )(wt)


def _make_lookup(n_rows: int, vocab_pad: int):
    info = plsc.get_sparse_core_info()
    nc, ns = info.num_cores, info.num_subcores
    nw = nc * ns
    rows_per_w = n_rows // nw
    iters = rows_per_w // K
    assert rows_per_w % K == 0

    mesh = plsc.VectorSubcoreMesh(core_axis_name="c", subcore_axis_name="s")

    @functools.partial(
        pl.kernel,
        mesh=mesh,
        out_type=jax.ShapeDtypeStruct((n_rows * L, D), jnp.float32),
        scratch_types=[
            pltpu.VMEM((K, L), jnp.int32),
            pltpu.VMEM((K * L, D), jnp.float32),
            pltpu.SemaphoreType.DMA,
        ],
        compiler_params=pltpu.CompilerParams(use_tc_tiling_on_sc=False),
    )
    def lookup(idx_hbm, table_hbm, out_hbm, idx_v, rows_v, sem):
        wid = lax.axis_index("s") * nc + lax.axis_index("c")
        row_base = wid * rows_per_w

        def body(g, carry):
            base = row_base + g * K
            pltpu.sync_copy(idx_hbm.at[pl.ds(base, K)], idx_v)
            copies = [
                pltpu.async_copy(
                    table_hbm.at[idx_v.at[j]],
                    rows_v.at[pl.ds(j * L, L)],
                    sem,
                )
                for j in range(K)
            ]
            for cp in copies:
                cp.wait()
            pltpu.sync_copy(rows_v, out_hbm.at[pl.ds(base * L, K * L)])
            return carry

        lax.fori_loop(0, iters, body, 0)

    return lookup


def _tc_to_tiles(lin, nh, nb):
    tg = 4
    lin4 = lin.reshape(nh, nb, D, 2 * D)

    def body(in_ref, out_ref):
        for u in range(tg):
            a = in_ref[0, u]
            t = a.T
            tt = jnp.concatenate([t[0:D], t[D : 2 * D]], axis=1)
            out_ref[0, :, u, :, :] = tt.reshape(8, 8, L)

    return pl.pallas_call(
        body,
        grid=(nh, nb // tg),
        in_specs=[pl.BlockSpec((1, tg, D, 2 * D), lambda h, t: (h, t, 0, 0))],
        out_specs=pl.BlockSpec((1, 8, tg, 8, L), lambda h, t: (h, 0, t, 0, 0)),
        out_shape=jax.ShapeDtypeStruct((nh, 8, nb, 8, L), jnp.float32),
    )(lin4)


def kernel(x, word_vectors):
    b, h = x.shape
    vocab, d = word_vectors.shape
    n = b * h
    nb = b // L
    half = TBLK // 2
    xi = x.astype(jnp.int32)
    remapped = (xi // TBLK) * TBLK + 2 * (xi % half) + (xi // half) % 2
    xperm = (
        remapped.T.reshape(h, nb, 2, D)
        .transpose(0, 1, 3, 2)
        .reshape(n // L, L)
    )
    wlin = _tc_transpose(word_vectors.T)
    vocab_pad = wlin.shape[0] * 2
    out = _make_lookup(n // L, vocab_pad)(xperm, wlin.reshape(vocab_pad, d))
    out5 = _tc_to_tiles(out.reshape(n // 2, 2 * d), h, nb)
    return out5.transpose(2, 4, 0, 1, 3).reshape(b, h, d)


# speedup vs baseline: 1.6087x; 1.5139x over previous
import functools

import jax
import jax.numpy as jnp
from jax import lax
from jax.experimental import pallas as pl
from jax.experimental.pallas import tpu as pltpu
from jax.experimental.pallas import tpu_sc as plsc

D = 64
L = 128
K = 4
TBLK = 8192


def _tc_transpose(wt):
    d, v = wt.shape
    grid = (v + TBLK - 1) // TBLK

    def body(in_ref, out_ref):
        a = in_ref[...].T
        out_ref[:, 0:d] = a[0 : TBLK // 2, :]
        out_ref[:, d : 2 * d] = a[TBLK // 2 : TBLK, :]

    return pl.pallas_call(
        body,
        grid=(grid,),
        in_specs=[pl.BlockSpec((d, TBLK), lambda g: (0, g))],
        out_specs=pl.BlockSpec((TBLK // 2, 2 * d), lambda g: (g, 0)),
        out_shape=jax.ShapeDtypeStruct((grid * TBLK // 2, 2 * d), jnp.float32),
    )(wt)


def _make_lookup(n_rows: int, vocab_pad: int):
    info = plsc.get_sparse_core_info()
    nc, ns = info.num_cores, info.num_subcores
    nw = nc * ns
    rows_per_w = n_rows // nw
    iters = rows_per_w // K
    assert rows_per_w % K == 0

    mesh = plsc.VectorSubcoreMesh(core_axis_name="c", subcore_axis_name="s")

    @functools.partial(
        pl.kernel,
        mesh=mesh,
        out_type=jax.ShapeDtypeStruct((n_rows * L, D), jnp.float32),
        scratch_types=[
            pltpu.VMEM((K, L), jnp.int32),
            pltpu.VMEM((K * L, D), jnp.float32),
            pltpu.SemaphoreType.DMA,
        ],
        compiler_params=pltpu.CompilerParams(use_tc_tiling_on_sc=False),
    )
    def lookup(idx_hbm, table_hbm, out_hbm, idx_v, rows_v, sem):
        wid = lax.axis_index("s") * nc + lax.axis_index("c")
        row_base = wid * rows_per_w

        def body(g, carry):
            base = row_base + g * K
            pltpu.sync_copy(idx_hbm.at[pl.ds(base, K)], idx_v)
            copies = [
                pltpu.async_copy(
                    table_hbm.at[idx_v.at[j]],
                    rows_v.at[pl.ds(j * L, L)],
                    sem,
                )
                for j in range(K)
            ]
            for cp in copies:
                cp.wait()
            pltpu.sync_copy(rows_v, out_hbm.at[pl.ds(base * L, K * L)])
            return carry

        lax.fori_loop(0, iters, body, 0)

    return lookup


def kernel(x, word_vectors):
    b, h = x.shape
    vocab, d = word_vectors.shape
    n = b * h
    half = TBLK // 2
    xi = x.reshape(-1).astype(jnp.int32)
    remapped = (xi // TBLK) * TBLK + 2 * (xi % half) + (xi // half) % 2
    idx2d = remapped.reshape(n // L, L)
    wlin = _tc_transpose(word_vectors.T)
    vocab_pad = wlin.shape[0] * 2
    out = _make_lookup(n // L, vocab_pad)(idx2d, wlin.reshape(vocab_pad, d))
    return out.reshape(b, h, d)
